# baseline (speedup 1.0000x reference)
import jax
import jax.numpy as jnp
from jax import lax
from jax.experimental import pallas as pl
from jax.experimental.pallas import tpu as pltpu
from jax.experimental.pallas import tpu_sc as plsc

_G = 512
_D = 256
_W = 128
_GA = _G + 8
_N = 50000
_BLK = 80
_NBLK = _N // _BLK
_NS = 16


def _seg_sum_body(x_hbm, ids_hbm, out_hbm, acc,
                  buf0, buf1, idx0, idx1, sem0, sem1):
    c = lax.axis_index("c")
    s = lax.axis_index("s")
    w = c * _NS + s
    lane = lax.iota(jnp.int32, 16)

    zrow = jnp.zeros((16,), jnp.float32)

    nw = (_NBLK - s + _NS - 1) // _NS

    def start(t, bufp, idxp, semp):
        r0 = (s + t * _NS) * _BLK
        pltpu.async_copy(
            x_hbm.at[pl.ds(r0, _BLK), pl.ds(c * _W, _W)], bufp, semp)
        pltpu.async_copy(ids_hbm.at[pl.ds(r0, _BLK)], idxp, semp)

    def flush(prev, runlen, avs):
        for j in range(_W // 16):
            plsc.addupdate(acc.at[prev, pl.ds(j * 16, 16)], avs[j])
        crow = _G + (prev >> 7)
        ccol0 = ((prev & 127) >> 4) * 16
        cvec = acc[crow, pl.ds(ccol0, 16)]
        acc[crow, pl.ds(ccol0, 16)] = cvec + jnp.where(
            lane == (prev & 15), runlen.astype(jnp.float32), 0.0)

    def process(bufp, idxp):
        first = idxp[pl.ds(0, 16)][0]
        last = idxp[pl.ds(_BLK - 16, 16)][15]
        block_uniform = first == last

        @pl.when(block_uniform)
        def _():
            avs = [jnp.zeros((16,), jnp.float32) for _ in range(_W // 16)]
            for k in range(_BLK):
                for j in range(_W // 16):
                    avs[j] = avs[j] + bufp[k, pl.ds(j * 16, 16)]
            flush(first, jnp.int32(_BLK), avs)

        def group(g, _):
            idchunk = idxp[pl.ds(g * 16, 16)]
            prev = idchunk[0]
            uniform = prev == idchunk[15]

            @pl.when(uniform)
            def _():
                avs = [jnp.zeros((16,), jnp.float32) for _ in range(_W // 16)]
                for k in range(16):
                    for j in range(_W // 16):
                        avs[j] = avs[j] + bufp[g * 16 + k, pl.ds(j * 16, 16)]
                flush(prev, jnp.int32(16), avs)

            @pl.when(jnp.logical_not(uniform))
            def _():
                pv = prev
                runlen = jnp.int32(0)
                avs = [jnp.zeros((16,), jnp.float32) for _ in range(_W // 16)]
                for k in range(16):
                    idk = idchunk[k]
                    changed = idk != pv

                    @pl.when(changed)
                    def _(pv=pv, runlen=runlen, avs=tuple(avs)):
                        flush(pv, runlen, list(avs))

                    for j in range(_W // 16):
                        vals = bufp[g * 16 + k, pl.ds(j * 16, 16)]
                        avs[j] = jnp.where(changed, vals, avs[j] + vals)
                    runlen = jnp.where(changed, 1, runlen + 1)
                    pv = idk
                flush(pv, runlen, avs)
            return 0

        @pl.when(jnp.logical_not(block_uniform))
        def _():
            lax.fori_loop(0, _BLK // 16, group, 0)

    start(0, buf0, idx0, sem0)

    def zero_acc(t, _):
        for j in range(8):
            acc[t, pl.ds(j * 16, 16)] = zrow
        return 0
    lax.fori_loop(0, _GA, zero_acc, 0)
    rings = ((buf0, idx0, sem0), (buf1, idx1, sem1))

    def pairbody(u, _):
        for pty in range(2):
            t = 2 * u + pty
            bufp, idxp, semp = rings[pty]
            nbufp, nidxp, nsemp = rings[1 - pty]

            @pl.when(t < nw)
            def _():
                @pl.when(t + 1 < nw)
                def _():
                    start(t + 1, nbufp, nidxp, nsemp)
                pltpu.make_async_copy(
                    x_hbm.at[pl.ds(0, _BLK), pl.ds(0, _W)], bufp, semp).wait()
                pltpu.make_async_copy(
                    ids_hbm.at[pl.ds(0, _BLK)], idxp, semp).wait()
                process(bufp, idxp)
        return 0
    lax.fori_loop(0, (nw + 1) // 2, pairbody, 0)

    pltpu.sync_copy(acc, out_hbm.at[w])


_seg_sum = pl.kernel(
    _seg_sum_body,
    out_type=jax.ShapeDtypeStruct((2 * _NS, _GA, _W), jnp.float32),
    mesh=plsc.VectorSubcoreMesh(core_axis_name="c", subcore_axis_name="s"),
    scratch_types=[
        pltpu.VMEM((_GA, _W), jnp.float32),
        pltpu.VMEM((_BLK, _W), jnp.float32),
        pltpu.VMEM((_BLK, _W), jnp.float32),
        pltpu.VMEM((_BLK,), jnp.int32),
        pltpu.VMEM((_BLK,), jnp.int32),
        pltpu.SemaphoreType.DMA,
        pltpu.SemaphoreType.DMA,
    ],
    name="segment_sum_sc",
)


def _mlp_body(p_ref, wenc_ref, benc_ref, w1_ref, b1_ref,
              w2_ref, b2_ref, out_ref):
    lo = jnp.sum(p_ref[0:_NS, 0:_G, :], axis=0)
    hi = jnp.sum(p_ref[_NS:, 0:_G, :], axis=0)
    seg = jnp.concatenate([lo, hi], axis=1)
    ctail = jnp.sum(p_ref[0:_NS, _G:_G + 4, :], axis=0)
    eye = (lax.broadcasted_iota(jnp.int32, (_W, _W), 0)
           == lax.broadcasted_iota(jnp.int32, (_W, _W), 1)).astype(jnp.float32)
    ones_col = jnp.ones((_W, 1), jnp.float32)
    cols = [
        jnp.dot(ctail[r:r + 1, :] * eye, ones_col,
                preferred_element_type=jnp.float32)
        for r in range(4)
    ]
    counts = jnp.concatenate(cols, axis=0)
    g = (jnp.dot(seg, wenc_ref[...], preferred_element_type=jnp.float32)
         + counts * benc_ref[...])
    h = jnp.maximum(
        jnp.dot(g, w1_ref[...], preferred_element_type=jnp.float32)
        + b1_ref[...], 0.0)
    out_ref[...] = (jnp.dot(h, w2_ref[...], preferred_element_type=jnp.float32)
                    + b2_ref[...])


_mlp = pl.pallas_call(
    _mlp_body,
    out_shape=jax.ShapeDtypeStruct((_G, _D), jnp.float32),
)


@jax.jit
def kernel(x, batch_ids, W_enc, b_enc, W1, b1, W2, b2):
    ids = batch_ids.astype(jnp.int32)
    partials = _seg_sum(x, ids)
    return _mlp(partials, W_enc, b_enc, W1, b1, W2, b2)


# speedup vs baseline: 1.6433x; 1.6433x over previous
import jax
import jax.numpy as jnp
from jax import lax
from jax.experimental import pallas as pl
from jax.experimental.pallas import tpu as pltpu
from jax.experimental.pallas import tpu_sc as plsc

_G = 512
_D = 256
_W = 128
_GA = _G + 8
_N = 50000
_BLK = 80
_NBLK = _N // _BLK
_NS = 16


def _seg_sum_body(x_hbm, ids_hbm, out_hbm, acc,
                  buf0, buf1, idx0, idx1, sem0, sem1):
    c = lax.axis_index("c")
    s = lax.axis_index("s")
    w = c * _NS + s
    lane = lax.iota(jnp.int32, 16)

    zrow = jnp.zeros((16,), jnp.float32)

    nw = (_NBLK - s + _NS - 1) // _NS

    def start(t, bufp, idxp, semp):
        r0 = (s + t * _NS) * _BLK
        pltpu.async_copy(
            x_hbm.at[pl.ds(r0, _BLK), pl.ds(c * _W, _W)], bufp, semp)
        pltpu.async_copy(ids_hbm.at[pl.ds(r0, _BLK)], idxp, semp)

    def flush(prev, runlen, avs):
        for j in range(_W // 16):
            plsc.addupdate(acc.at[prev, pl.ds(j * 16, 16)], avs[j])
        crow = _G + (prev >> 7)
        ccol0 = ((prev & 127) >> 4) * 16
        cvec = acc[crow, pl.ds(ccol0, 16)]
        acc[crow, pl.ds(ccol0, 16)] = cvec + jnp.where(
            lane == (prev & 15), runlen.astype(jnp.float32), 0.0)

    def process(bufp, idxp):
        def group(g, _):
            idchunk = idxp[pl.ds(g * 16, 16)]
            prev = idchunk[0]
            uniform = prev == idchunk[15]

            @pl.when(uniform)
            def _():
                avs = [jnp.zeros((16,), jnp.float32) for _ in range(_W // 16)]
                for k in range(16):
                    for j in range(_W // 16):
                        avs[j] = avs[j] + bufp[g * 16 + k, pl.ds(j * 16, 16)]
                flush(prev, jnp.int32(16), avs)

            @pl.when(jnp.logical_not(uniform))
            def _():
                pv = prev
                runlen = jnp.int32(0)
                avs = [jnp.zeros((16,), jnp.float32) for _ in range(_W // 16)]
                for k in range(16):
                    idk = idchunk[k]
                    changed = idk != pv

                    @pl.when(changed)
                    def _(pv=pv, runlen=runlen, avs=tuple(avs)):
                        flush(pv, runlen, list(avs))

                    for j in range(_W // 16):
                        vals = bufp[g * 16 + k, pl.ds(j * 16, 16)]
                        avs[j] = jnp.where(changed, vals, avs[j] + vals)
                    runlen = jnp.where(changed, 1, runlen + 1)
                    pv = idk
                flush(pv, runlen, avs)
            return 0
        lax.fori_loop(0, _BLK // 16, group, 0)

    start(0, buf0, idx0, sem0)

    def zero_acc(t, _):
        for j in range(8):
            acc[t, pl.ds(j * 16, 16)] = zrow
        return 0
    lax.fori_loop(0, _GA, zero_acc, 0)
    rings = ((buf0, idx0, sem0), (buf1, idx1, sem1))

    def pairbody(u, _):
        for pty in range(2):
            t = 2 * u + pty
            bufp, idxp, semp = rings[pty]
            nbufp, nidxp, nsemp = rings[1 - pty]

            @pl.when(t < nw)
            def _():
                @pl.when(t + 1 < nw)
                def _():
                    start(t + 1, nbufp, nidxp, nsemp)
                pltpu.make_async_copy(
                    x_hbm.at[pl.ds(0, _BLK), pl.ds(0, _W)], bufp, semp).wait()
                pltpu.make_async_copy(
                    ids_hbm.at[pl.ds(0, _BLK)], idxp, semp).wait()
                process(bufp, idxp)
        return 0
    lax.fori_loop(0, (nw + 1) // 2, pairbody, 0)

    pltpu.sync_copy(acc, out_hbm.at[w])


_seg_sum = pl.kernel(
    _seg_sum_body,
    out_type=jax.ShapeDtypeStruct((2 * _NS, _GA, _W), jnp.float32),
    mesh=plsc.VectorSubcoreMesh(core_axis_name="c", subcore_axis_name="s"),
    scratch_types=[
        pltpu.VMEM((_GA, _W), jnp.float32),
        pltpu.VMEM((_BLK, _W), jnp.float32),
        pltpu.VMEM((_BLK, _W), jnp.float32),
        pltpu.VMEM((_BLK,), jnp.int32),
        pltpu.VMEM((_BLK,), jnp.int32),
        pltpu.SemaphoreType.DMA,
        pltpu.SemaphoreType.DMA,
    ],
    name="segment_sum_sc",
)


def _mlp_body(p_ref, wenc_ref, benc_ref, w1_ref, b1_ref,
              w2_ref, b2_ref, out_ref):
    lo = jnp.sum(p_ref[0:_NS, 0:_G, :], axis=0)
    hi = jnp.sum(p_ref[_NS:, 0:_G, :], axis=0)
    seg = jnp.concatenate([lo, hi], axis=1)
    ctail = jnp.sum(p_ref[0:_NS, _G:_G + 4, :], axis=0)
    eye = (lax.broadcasted_iota(jnp.int32, (_W, _W), 0)
           == lax.broadcasted_iota(jnp.int32, (_W, _W), 1)).astype(jnp.float32)
    ones_col = jnp.ones((_W, 1), jnp.float32)
    cols = [
        jnp.dot(ctail[r:r + 1, :] * eye, ones_col,
                preferred_element_type=jnp.float32)
        for r in range(4)
    ]
    counts = jnp.concatenate(cols, axis=0)
    g = (jnp.dot(seg, wenc_ref[...], preferred_element_type=jnp.float32)
         + counts * benc_ref[...])
    h = jnp.maximum(
        jnp.dot(g, w1_ref[...], preferred_element_type=jnp.float32)
        + b1_ref[...], 0.0)
    out_ref[...] = (jnp.dot(h, w2_ref[...], preferred_element_type=jnp.float32)
                    + b2_ref[...])


_mlp = pl.pallas_call(
    _mlp_body,
    out_shape=jax.ShapeDtypeStruct((_G, _D), jnp.float32),
)


@jax.jit
def kernel(x, batch_ids, W_enc, b_enc, W1, b1, W2, b2):
    ids = batch_ids.astype(jnp.int32)
    partials = _seg_sum(x, ids)
    return _mlp(partials, W_enc, b_enc, W1, b1, W2, b2)


# speedup vs baseline: 1.6456x; 1.0014x over previous
import jax
import jax.numpy as jnp
from jax import lax
from jax.experimental import pallas as pl
from jax.experimental.pallas import tpu as pltpu
from jax.experimental.pallas import tpu_sc as plsc

_G = 512
_D = 256
_W = 128
_GA = _G + 8
_N = 50000
_BLK = 80
_NBLK = _N // _BLK
_NS = 16


def _seg_sum_body(x_hbm, ids_hbm, out_hbm, acc,
                  buf0, buf1, idx0, idx1, sem0, sem1):
    c = lax.axis_index("c")
    s = lax.axis_index("s")
    w = c * _NS + s
    lane = lax.iota(jnp.int32, 16)

    zrow = jnp.zeros((16,), jnp.float32)

    nw = (_NBLK - s + _NS - 1) // _NS

    def start(t, bufp, idxp, semp):
        r0 = (s + t * _NS) * _BLK
        pltpu.async_copy(
            x_hbm.at[pl.ds(r0, _BLK), pl.ds(c * _W, _W)], bufp, semp)
        pltpu.async_copy(ids_hbm.at[pl.ds(r0, _BLK)], idxp, semp)

    def flush(prev, runlen, avs):
        for j in range(_W // 16):
            plsc.addupdate(acc.at[prev, pl.ds(j * 16, 16)], avs[j])
        crow = _G + (prev >> 7)
        ccol0 = ((prev & 127) >> 4) * 16
        cvec = acc[crow, pl.ds(ccol0, 16)]
        acc[crow, pl.ds(ccol0, 16)] = cvec + jnp.where(
            lane == (prev & 15), runlen.astype(jnp.float32), 0.0)

    def process(bufp, idxp):
        def group(g, _):
            idchunk = idxp[pl.ds(g * 16, 16)]
            prev = idchunk[0]
            uniform = prev == idchunk[15]

            @pl.when(uniform)
            def _():
                avs = [jnp.zeros((16,), jnp.float32) for _ in range(_W // 16)]
                for k in range(16):
                    for j in range(_W // 16):
                        avs[j] = avs[j] + bufp[g * 16 + k, pl.ds(j * 16, 16)]
                flush(prev, jnp.int32(16), avs)

            @pl.when(jnp.logical_not(uniform))
            def _():
                pv = prev
                runlen = jnp.int32(0)
                avs = [jnp.zeros((16,), jnp.float32) for _ in range(_W // 16)]
                for k in range(16):
                    idk = idchunk[k]
                    changed = idk != pv

                    @pl.when(changed)
                    def _(pv=pv, runlen=runlen, avs=tuple(avs)):
                        flush(pv, runlen, list(avs))

                    for j in range(_W // 16):
                        vals = bufp[g * 16 + k, pl.ds(j * 16, 16)]
                        avs[j] = jnp.where(changed, vals, avs[j] + vals)
                    runlen = jnp.where(changed, 1, runlen + 1)
                    pv = idk
                flush(pv, runlen, avs)
            return 0
        lax.fori_loop(0, _BLK // 16, group, 0)

    start(0, buf0, idx0, sem0)

    def zero_acc(t, _):
        for r in range(2):
            for j in range(8):
                acc[t * 2 + r, pl.ds(j * 16, 16)] = zrow
        return 0
    lax.fori_loop(0, _GA // 2, zero_acc, 0)
    rings = ((buf0, idx0, sem0), (buf1, idx1, sem1))

    def pairbody(u, _):
        for pty in range(2):
            t = 2 * u + pty
            bufp, idxp, semp = rings[pty]
            nbufp, nidxp, nsemp = rings[1 - pty]

            @pl.when(t < nw)
            def _():
                @pl.when(t + 1 < nw)
                def _():
                    start(t + 1, nbufp, nidxp, nsemp)
                pltpu.make_async_copy(
                    x_hbm.at[pl.ds(0, _BLK), pl.ds(0, _W)], bufp, semp).wait()
                pltpu.make_async_copy(
                    ids_hbm.at[pl.ds(0, _BLK)], idxp, semp).wait()
                process(bufp, idxp)
        return 0
    lax.fori_loop(0, (nw + 1) // 2, pairbody, 0)

    pltpu.sync_copy(acc, out_hbm.at[w])


_seg_sum = pl.kernel(
    _seg_sum_body,
    out_type=jax.ShapeDtypeStruct((2 * _NS, _GA, _W), jnp.float32),
    mesh=plsc.VectorSubcoreMesh(core_axis_name="c", subcore_axis_name="s"),
    scratch_types=[
        pltpu.VMEM((_GA, _W), jnp.float32),
        pltpu.VMEM((_BLK, _W), jnp.float32),
        pltpu.VMEM((_BLK, _W), jnp.float32),
        pltpu.VMEM((_BLK,), jnp.int32),
        pltpu.VMEM((_BLK,), jnp.int32),
        pltpu.SemaphoreType.DMA,
        pltpu.SemaphoreType.DMA,
    ],
    name="segment_sum_sc",
)


def _mlp_body(p_ref, wenc_ref, benc_ref, w1_ref, b1_ref,
              w2_ref, b2_ref, out_ref):
    lo = jnp.sum(p_ref[0:_NS, 0:_G, :], axis=0)
    hi = jnp.sum(p_ref[_NS:, 0:_G, :], axis=0)
    seg = jnp.concatenate([lo, hi], axis=1)
    ctail = jnp.sum(p_ref[0:_NS, _G:_G + 4, :], axis=0)
    eye = (lax.broadcasted_iota(jnp.int32, (_W, _W), 0)
           == lax.broadcasted_iota(jnp.int32, (_W, _W), 1)).astype(jnp.float32)
    ones_col = jnp.ones((_W, 1), jnp.float32)
    cols = [
        jnp.dot(ctail[r:r + 1, :] * eye, ones_col,
                preferred_element_type=jnp.float32)
        for r in range(4)
    ]
    counts = jnp.concatenate(cols, axis=0)
    g = (jnp.dot(seg, wenc_ref[...], preferred_element_type=jnp.float32)
         + counts * benc_ref[...])
    h = jnp.maximum(
        jnp.dot(g, w1_ref[...], preferred_element_type=jnp.float32)
        + b1_ref[...], 0.0)
    out_ref[...] = (jnp.dot(h, w2_ref[...], preferred_element_type=jnp.float32)
                    + b2_ref[...])


_mlp = pl.pallas_call(
    _mlp_body,
    out_shape=jax.ShapeDtypeStruct((_G, _D), jnp.float32),
)


@jax.jit
def kernel(x, batch_ids, W_enc, b_enc, W1, b1, W2, b2):
    ids = batch_ids.astype(jnp.int32)
    partials = _seg_sum(x, ids)
    return _mlp(partials, W_enc, b_enc, W1, b1, W2, b2)
